# baseline (speedup 1.0000x reference)
import functools

import jax
import jax.numpy as jnp
from jax import lax
from jax.experimental import pallas as pl
from jax.experimental.pallas import tpu as pltpu
from jax.experimental.pallas import tpu_sc as plsc

_VOCAB = 119
_VPAD = 128
_D = 512
_BATCH, _SEQ = 4096, 50
_SEQ_SC = 48
_SEQ_TC = _SEQ - _SEQ_SC

_NC, _NS = 2, 16
_NW = _NC * _NS
_B_PER_W = _BATCH // _NW

_FIX_GRP = 64


def _proj_body(t1, w1, b1, t2, w2, b2, t3, w3, b3, o1, o2, o3):
    o1[...] = jnp.dot(t1[...], w1[...], preferred_element_type=jnp.float32) + b1[...]
    o2[...] = jnp.dot(t2[...], w2[...], preferred_element_type=jnp.float32) + b2[...]
    o3[...] = jnp.dot(t3[...], w3[...], preferred_element_type=jnp.float32) + b3[...]


def _project_tables(t1, w1, b1, t2, w2, b2, t3, w3, b3):
    out = [jax.ShapeDtypeStruct((_VOCAB, _D), jnp.float32)] * 3
    return pl.pallas_call(_proj_body, out_shape=out)(
        t1, w1, b1.reshape(1, _D), t2, w2, b2.reshape(1, _D),
        t3, w3, b3.reshape(1, _D))


_mesh = plsc.VectorSubcoreMesh(core_axis_name="c", subcore_axis_name="s")


_RING = 4


@functools.partial(
    pl.kernel,
    mesh=_mesh,
    out_type=jax.ShapeDtypeStruct((_BATCH, _SEQ, _D), jnp.float32),
    scratch_types=[
        pltpu.VMEM((_B_PER_W, _SEQ_SC), jnp.int32),
        pltpu.VMEM((_SEQ_SC, _D), jnp.float32),
        pltpu.VMEM((_SEQ_SC, _D), jnp.float32),
        pltpu.VMEM((_SEQ_SC, _D), jnp.float32),
        pltpu.VMEM((_SEQ_SC, _D), jnp.float32),
        pltpu.SemaphoreType.DMA,
        pltpu.SemaphoreType.DMA,
    ],
    compiler_params=pltpu.CompilerParams(use_tc_tiling_on_sc=True),
)
def _gather_one(tbl, idx_hbm, out, idx_v, r0, r1, r2, r3, gsem, wsem):
    wid = lax.axis_index("s") * _NC + lax.axis_index("c")
    bufs = (r0, r1, r2, r3)

    b0 = wid * _B_PER_W
    pltpu.sync_copy(idx_hbm.at[pl.ds(b0, _B_PER_W)], idx_v)

    def _wait_gather(j):
        pltpu.make_async_copy(
            tbl.at[pl.ds(0, _SEQ_SC)], bufs[j], gsem).wait()

    def _drain_write(j):
        pltpu.make_async_copy(
            bufs[j], out.at[0, pl.ds(0, _SEQ_SC)], wsem).wait()

    def ring_body(c, carry):
        for j in range(_RING):
            b = c * _RING + j
            @pl.when(c > 0)
            def _ring_guard():
                _drain_write(j)

            pltpu.async_copy(tbl.at[idx_v.at[b]], bufs[j], gsem)
            if j > 0:
                _wait_gather(j - 1)
                pltpu.async_copy(
                    bufs[j - 1], out.at[b0 + b - 1, pl.ds(0, _SEQ_SC)],
                    wsem)
            else:
                @pl.when(c > 0)
                def _prev_iter():
                    _wait_gather(_RING - 1)
                    pltpu.async_copy(
                        bufs[_RING - 1],
                        out.at[b0 + b - 1, pl.ds(0, _SEQ_SC)], wsem)
        return carry

    lax.fori_loop(0, _B_PER_W // _RING, ring_body, 0)
    _wait_gather(_RING - 1)
    pltpu.async_copy(
        bufs[_RING - 1], out.at[b0 + _B_PER_W - 1, pl.ds(0, _SEQ_SC)], wsem)
    for j in range(_RING):
        _drain_write(j)


def _fix_body(zb, p1, p2, p3, o1, o2, o3):
    zcol = zb[0].reshape(_FIX_GRP * _SEQ_TC, 1)
    iota_v = lax.broadcasted_iota(jnp.int32, (_FIX_GRP * _SEQ_TC, _VPAD), 1)
    oh = (zcol == iota_v).astype(jnp.float32)
    for p, o in ((p1, o1), (p2, o2), (p3, o3)):
        r = jnp.dot(oh, p[...], preferred_element_type=jnp.float32)
        o[...] = r.reshape(_FIX_GRP, _SEQ_TC, _D)


def _fix_tails(zfix, p1, p2, p3):
    ngrp = _BATCH // _FIX_GRP
    z3 = zfix.reshape(ngrp, 1, _FIX_GRP * _SEQ_TC)
    pp = [jnp.pad(p, ((0, _VPAD - _VOCAB), (0, 0))) for p in (p1, p2, p3)]
    out_spec = pl.BlockSpec((_FIX_GRP, _SEQ_TC, _D), lambda i: (i, 0, 0))
    return pl.pallas_call(
        _fix_body,
        grid=(ngrp,),
        in_specs=[
            pl.BlockSpec((1, 1, _FIX_GRP * _SEQ_TC), lambda i: (i, 0, 0)),
            pl.BlockSpec((_VPAD, _D), lambda i: (0, 0)),
            pl.BlockSpec((_VPAD, _D), lambda i: (0, 0)),
            pl.BlockSpec((_VPAD, _D), lambda i: (0, 0)),
        ],
        out_specs=[out_spec] * 3,
        out_shape=[jax.ShapeDtypeStruct((_BATCH, _SEQ_TC, _D), jnp.float32)] * 3,
    )(z3, pp[0], pp[1], pp[2])


def kernel(Z, table_mat2vec, table_magpie, table_oliynyk,
           W_mat2vec, b_mat2vec, W_magpie, b_magpie, W_oliynyk, b_oliynyk):
    p1, p2, p3 = _project_tables(
        table_mat2vec, W_mat2vec, b_mat2vec,
        table_magpie, W_magpie, b_magpie,
        table_oliynyk, W_oliynyk, b_oliynyk)
    zs = Z[:, :_SEQ_SC]
    t1, t2, t3 = _fix_tails(Z[:, _SEQ_SC:], p1, p2, p3)
    r1 = lax.dynamic_update_slice(_gather_one(p1, zs), t1, (0, _SEQ_SC, 0))
    p2b, zs2, r1b = lax.optimization_barrier((p2, zs, r1))
    r2 = lax.dynamic_update_slice(_gather_one(p2b, zs2), t2, (0, _SEQ_SC, 0))
    p3b, zs3, r2b = lax.optimization_barrier((p3, zs, r2))
    r3 = lax.dynamic_update_slice(_gather_one(p3b, zs3), t3, (0, _SEQ_SC, 0))
    return (r1b, r2b, r3)


# speedup vs baseline: 1.3526x; 1.3526x over previous
import functools

import jax
import jax.numpy as jnp
from jax import lax
from jax.experimental import pallas as pl
from jax.experimental.pallas import tpu as pltpu
from jax.experimental.pallas import tpu_sc as plsc

_VOCAB = 119
_VPAD = 128
_D = 512
_BATCH, _SEQ = 4096, 50
_SEQ_SC = 48
_SEQ_TC = _SEQ - _SEQ_SC

_NC, _NS = 2, 16
_NW = _NC * _NS
_B_PER_W = _BATCH // _NW

_FIX_GRP = 64


def _proj_body(t1, w1, b1, t2, w2, b2, t3, w3, b3, o1, o2, o3):
    o1[...] = jnp.dot(t1[...], w1[...], preferred_element_type=jnp.float32) + b1[...]
    o2[...] = jnp.dot(t2[...], w2[...], preferred_element_type=jnp.float32) + b2[...]
    o3[...] = jnp.dot(t3[...], w3[...], preferred_element_type=jnp.float32) + b3[...]


def _project_tables(t1, w1, b1, t2, w2, b2, t3, w3, b3):
    out = [jax.ShapeDtypeStruct((_VOCAB, _D), jnp.float32)] * 3
    return pl.pallas_call(_proj_body, out_shape=out)(
        t1, w1, b1.reshape(1, _D), t2, w2, b2.reshape(1, _D),
        t3, w3, b3.reshape(1, _D))


_mesh = plsc.VectorSubcoreMesh(core_axis_name="c", subcore_axis_name="s")


@functools.partial(
    pl.kernel,
    mesh=_mesh,
    out_type=[jax.ShapeDtypeStruct((_BATCH, _SEQ, _D), jnp.float32)] * 3,
    scratch_types=[
        pltpu.VMEM((_B_PER_W, _SEQ_SC), jnp.int32),
        pltpu.VMEM((_SEQ_SC, _D), jnp.float32),
        pltpu.VMEM((_SEQ_SC, _D), jnp.float32),
        pltpu.VMEM((_SEQ_SC, _D), jnp.float32),
        pltpu.SemaphoreType.DMA,
        pltpu.SemaphoreType.DMA,
    ],
    compiler_params=pltpu.CompilerParams(use_tc_tiling_on_sc=True),
)
def _gather_sc(p1, p2, p3, idx_hbm, o1, o2, o3,
               idx_v, r0, r1, r2, gsem, wsem):
    wid = lax.axis_index("s") * _NC + lax.axis_index("c")
    tbls = (p1, p2, p3)
    outs = (o1, o2, o3)
    bufs = (r0, r1, r2)

    b0 = wid * _B_PER_W
    pltpu.sync_copy(idx_hbm.at[pl.ds(b0, _B_PER_W)], idx_v)

    def _wait_gather(t):
        pltpu.make_async_copy(
            tbls[0].at[pl.ds(0, _SEQ_SC)], bufs[t], gsem).wait()

    def _drain_write(t):
        pltpu.make_async_copy(
            bufs[t], outs[0].at[0, pl.ds(0, _SEQ_SC)], wsem).wait()

    def batch_body(b, carry):
        for t in range(3):
            @pl.when(b > 0)
            def _ring_guard():
                _drain_write(t)

            pltpu.async_copy(tbls[t].at[idx_v.at[b]], bufs[t], gsem)
            if t > 0:
                _wait_gather(t - 1)
                pltpu.async_copy(
                    bufs[t - 1], outs[t - 1].at[b0 + b, pl.ds(0, _SEQ_SC)],
                    wsem)
            else:
                @pl.when(b > 0)
                def _prev_iter():
                    _wait_gather(2)
                    pltpu.async_copy(
                        bufs[2], outs[2].at[b0 + b - 1, pl.ds(0, _SEQ_SC)],
                        wsem)
        return carry

    lax.fori_loop(0, _B_PER_W, batch_body, 0)
    _wait_gather(2)
    pltpu.async_copy(
        bufs[2], outs[2].at[b0 + _B_PER_W - 1, pl.ds(0, _SEQ_SC)], wsem)
    for t in range(3):
        _drain_write(t)


def _fix_body(zb, p1, p2, p3, o1, o2, o3):
    zcol = zb[0].reshape(_FIX_GRP * _SEQ_TC, 1)
    iota_v = lax.broadcasted_iota(jnp.int32, (_FIX_GRP * _SEQ_TC, _VPAD), 1)
    oh = (zcol == iota_v).astype(jnp.float32)
    for p, o in ((p1, o1), (p2, o2), (p3, o3)):
        r = jnp.dot(oh, p[...], preferred_element_type=jnp.float32)
        o[...] = r.reshape(_FIX_GRP, _SEQ_TC, _D)


def _fix_tails(zfix, p1, p2, p3):
    ngrp = _BATCH // _FIX_GRP
    z3 = zfix.reshape(ngrp, 1, _FIX_GRP * _SEQ_TC)
    pp = [jnp.pad(p, ((0, _VPAD - _VOCAB), (0, 0))) for p in (p1, p2, p3)]
    out_spec = pl.BlockSpec((_FIX_GRP, _SEQ_TC, _D), lambda i: (i, 0, 0))
    return pl.pallas_call(
        _fix_body,
        grid=(ngrp,),
        in_specs=[
            pl.BlockSpec((1, 1, _FIX_GRP * _SEQ_TC), lambda i: (i, 0, 0)),
            pl.BlockSpec((_VPAD, _D), lambda i: (0, 0)),
            pl.BlockSpec((_VPAD, _D), lambda i: (0, 0)),
            pl.BlockSpec((_VPAD, _D), lambda i: (0, 0)),
        ],
        out_specs=[out_spec] * 3,
        out_shape=[jax.ShapeDtypeStruct((_BATCH, _SEQ_TC, _D), jnp.float32)] * 3,
    )(z3, pp[0], pp[1], pp[2])


def kernel(Z, table_mat2vec, table_magpie, table_oliynyk,
           W_mat2vec, b_mat2vec, W_magpie, b_magpie, W_oliynyk, b_oliynyk):
    p1, p2, p3 = _project_tables(
        table_mat2vec, W_mat2vec, b_mat2vec,
        table_magpie, W_magpie, b_magpie,
        table_oliynyk, W_oliynyk, b_oliynyk)
    o1, o2, o3 = _gather_sc(p1, p2, p3, Z[:, :_SEQ_SC])
    t1, t2, t3 = _fix_tails(Z[:, _SEQ_SC:], p1, p2, p3)
    return tuple(
        lax.dynamic_update_slice(o, t, (0, _SEQ_SC, 0))
        for o, t in ((o1, t1), (o2, t2), (o3, t3)))
